# baseline (speedup 1.0000x reference)
import functools

import jax
import jax.numpy as jnp
from jax import lax
from jax.experimental import pallas as pl
from jax.experimental.pallas import tpu as pltpu
from jax.experimental.pallas import tpu_sc as plsc

T = 4096
D = 768
E = 8
K = 2
F = 3072
C = 1280
S_TOT = E * C
TB = 512
NT = T // TB
CB = 256
FB = 768
NW = 32
LANES = 128


def _router_body(x_ref, wr_ref, d0_ref, d1_ref, c0_ref, c1_ref, g0_ref,
                 g1_ref, loss_ref, me_acc, cnt_acc):
    i = pl.program_id(0)

    @pl.when(i == 0)
    def _():
        me_acc[...] = jnp.zeros((8, LANES), jnp.float32)
        cnt_acc[...] = jnp.zeros((8, LANES), jnp.float32)

    x = x_ref[...]
    logits = jnp.dot(x, wr_ref[...], preferred_element_type=jnp.float32)
    lane = lax.broadcasted_iota(jnp.int32, (TB, LANES), 1)
    emask = lane < E
    lm = jnp.where(emask, logits, jnp.float32(-1e30))
    m = jnp.max(lm, axis=1, keepdims=True)
    ex = jnp.where(emask, jnp.exp(lm - m), 0.0)
    probs = ex / jnp.sum(ex, axis=1, keepdims=True)

    v1 = jnp.max(probs, axis=1, keepdims=True)
    i1 = jnp.min(jnp.where(probs == v1, lane, LANES), axis=1, keepdims=True)
    oh1 = (lane == i1).astype(jnp.float32)
    probs2 = jnp.where(lane == i1, -1.0, probs)
    v2 = jnp.max(probs2, axis=1, keepdims=True)
    i2 = jnp.min(jnp.where(probs2 == v2, lane, LANES), axis=1, keepdims=True)
    oh2 = (lane == i2).astype(jnp.float32)
    den = v1 + v2
    g1v = v1 / den
    g2v = v2 / den

    oh = oh1 + oh2
    r_io = lax.broadcasted_iota(jnp.int32, (TB, TB), 0)
    c_io = lax.broadcasted_iota(jnp.int32, (TB, TB), 1)
    tri = (r_io > c_io).astype(jnp.float32)
    pref = jnp.dot(tri, oh, preferred_element_type=jnp.float32)
    carry = cnt_acc[0:1, :]
    p_all = carry + pref
    pos0 = jnp.sum(jnp.where(lane == i1, p_all, 0.0), axis=1, keepdims=True)
    pos1 = jnp.sum(jnp.where(lane == i2, p_all, 0.0), axis=1, keepdims=True)
    pos0 = pos0.astype(jnp.int32)
    pos1 = pos1.astype(jnp.int32)
    keep0 = pos0 < C
    keep1 = pos1 < C
    slot0 = i1 * C + pos0
    slot1 = i2 * C + pos1

    d0 = jnp.where(keep0, slot0, S_TOT)
    d1 = jnp.where(keep1, slot1, S_TOT)
    c0 = jnp.where(keep0, slot0, 0)
    c1 = jnp.where(keep1, slot1, 0)
    cg0 = jnp.where(keep0, g1v, 0.0)
    cg1 = jnp.where(keep1, g2v, 0.0)

    d0_ref[...] = jnp.broadcast_to(d0, (TB, 8))
    d1_ref[...] = jnp.broadcast_to(d1, (TB, 8))
    c0_ref[...] = jnp.broadcast_to(c0, (TB, 8))
    c1_ref[...] = jnp.broadcast_to(c1, (TB, 8))
    g0_ref[...] = jnp.broadcast_to(cg0, (TB, 8))
    g1_ref[...] = jnp.broadcast_to(cg1, (TB, 8))

    cnt_acc[0:1, :] = carry + jnp.sum(oh, axis=0, keepdims=True)
    me_acc[0:1, :] = me_acc[0:1, :] + jnp.sum(probs, axis=0, keepdims=True)

    @pl.when(i == NT - 1)
    def _():
        me = me_acc[0:1, :] / T
        ce = cnt_acc[0:1, :] / (T * K)
        loss = E * jnp.sum(me * ce)
        loss_ref[...] = jnp.zeros((8, LANES), jnp.float32) + loss


def _router(x, wr_p):
    return pl.pallas_call(
        _router_body,
        grid=(NT,),
        in_specs=[
            pl.BlockSpec((TB, D), lambda i: (i, 0)),
            pl.BlockSpec((D, LANES), lambda i: (0, 0)),
        ],
        out_specs=[
            pl.BlockSpec((TB, 8), lambda i: (i, 0)),
            pl.BlockSpec((TB, 8), lambda i: (i, 0)),
            pl.BlockSpec((TB, 8), lambda i: (i, 0)),
            pl.BlockSpec((TB, 8), lambda i: (i, 0)),
            pl.BlockSpec((TB, 8), lambda i: (i, 0)),
            pl.BlockSpec((TB, 8), lambda i: (i, 0)),
            pl.BlockSpec((8, LANES), lambda i: (0, 0)),
        ],
        out_shape=[
            jax.ShapeDtypeStruct((T, 8), jnp.int32),
            jax.ShapeDtypeStruct((T, 8), jnp.int32),
            jax.ShapeDtypeStruct((T, 8), jnp.int32),
            jax.ShapeDtypeStruct((T, 8), jnp.int32),
            jax.ShapeDtypeStruct((T, 8), jnp.float32),
            jax.ShapeDtypeStruct((T, 8), jnp.float32),
            jax.ShapeDtypeStruct((8, LANES), jnp.float32),
        ],
        scratch_shapes=[
            pltpu.VMEM((8, LANES), jnp.float32),
            pltpu.VMEM((8, LANES), jnp.float32),
        ],
    )(x, wr_p)


_INV_N = S_TOT + 128
_DG_PW = S_TOT // NW
_DG_CH = 80
_CG_PW = T // NW
_CG_CH = 64


@functools.cache
def _sc_kernels():
    mesh = plsc.VectorSubcoreMesh(core_axis_name="c", subcore_axis_name="s")

    @functools.partial(
        pl.kernel,
        mesh=mesh,
        out_type=jax.ShapeDtypeStruct((_INV_N,), jnp.int32),
        scratch_types=[
            pltpu.VMEM((T,), jnp.int32),
            pltpu.VMEM((T,), jnp.int32),
            pltpu.VMEM((_INV_N,), jnp.int32),
        ],
        compiler_params=pltpu.CompilerParams(needs_layout_passes=False),
    )
    def build_inv(d0_hbm, d1_hbm, inv_hbm, s0_v, s1_v, inv_v):
        wid = lax.axis_index("s") * 2 + lax.axis_index("c")

        @pl.when(wid == 0)
        def _():
            pltpu.sync_copy(d0_hbm, s0_v)
            pltpu.sync_copy(d1_hbm, s1_v)

            def init_body(j, carry):
                inv_v[pl.ds(j * 16, 16)] = jnp.zeros((16,), jnp.int32)
                return carry

            lax.fori_loop(0, _INV_N // 16, init_body, 0)

            def scat_body(j, carry):
                base = j * 16
                toks = lax.iota(jnp.int32, 16) + base
                plsc.store_scatter(inv_v, [s0_v[pl.ds(base, 16)]], toks)
                plsc.store_scatter(inv_v, [s1_v[pl.ds(base, 16)]], toks)
                return carry

            lax.fori_loop(0, T // 16, scat_body, 0)
            pltpu.sync_copy(inv_v, inv_hbm)

    @functools.partial(
        pl.kernel,
        mesh=mesh,
        out_type=jax.ShapeDtypeStruct((S_TOT, D), jnp.float32),
        scratch_types=[
            pltpu.VMEM((_DG_PW,), jnp.int32),
            pltpu.VMEM((_DG_CH, D), jnp.float32),
            pltpu.SemaphoreType.DMA,
        ],
    )
    def dispatch(x_hbm, inv_hbm, buf_hbm, idx_v, rows_v, sem):
        wid = lax.axis_index("s") * 2 + lax.axis_index("c")
        base = wid * _DG_PW
        pltpu.sync_copy(inv_hbm.at[pl.ds(base, _DG_PW)], idx_v)
        for ch in range(_DG_PW // _DG_CH):
            pltpu.async_copy(
                x_hbm.at[idx_v.at[pl.ds(ch * _DG_CH, _DG_CH)]], rows_v, sem
            ).wait()
            pltpu.sync_copy(
                rows_v, buf_hbm.at[pl.ds(base + ch * _DG_CH, _DG_CH)]
            )

    @functools.partial(
        pl.kernel,
        mesh=mesh,
        out_type=(
            jax.ShapeDtypeStruct((T, D), jnp.float32),
            jax.ShapeDtypeStruct((T, D), jnp.float32),
        ),
        scratch_types=[
            pltpu.VMEM((_CG_PW,), jnp.int32),
            pltpu.VMEM((_CG_PW,), jnp.int32),
            pltpu.VMEM((_CG_CH, D), jnp.float32),
            pltpu.SemaphoreType.DMA,
        ],
    )
    def combine_gather(out_hbm, c0_hbm, c1_hbm, r0_hbm, r1_hbm, i0_v, i1_v,
                       rows_v, sem):
        wid = lax.axis_index("s") * 2 + lax.axis_index("c")
        base = wid * _CG_PW
        pltpu.sync_copy(c0_hbm.at[pl.ds(base, _CG_PW)], i0_v)
        pltpu.sync_copy(c1_hbm.at[pl.ds(base, _CG_PW)], i1_v)
        for ch in range(_CG_PW // _CG_CH):
            pltpu.async_copy(
                out_hbm.at[i0_v.at[pl.ds(ch * _CG_CH, _CG_CH)]], rows_v, sem
            ).wait()
            pltpu.sync_copy(
                rows_v, r0_hbm.at[pl.ds(base + ch * _CG_CH, _CG_CH)]
            )
        for ch in range(_CG_PW // _CG_CH):
            pltpu.async_copy(
                out_hbm.at[i1_v.at[pl.ds(ch * _CG_CH, _CG_CH)]], rows_v, sem
            ).wait()
            pltpu.sync_copy(
                rows_v, r1_hbm.at[pl.ds(base + ch * _CG_CH, _CG_CH)]
            )

    return build_inv, dispatch, combine_gather


def _ffn_body(buf_ref, w1_ref, b1_ref, w2_ref, b2_ref, out_ref):
    e = pl.program_id(0)
    fb = pl.program_id(2)
    xb = buf_ref[...].astype(jnp.bfloat16)
    w1 = w1_ref[0].astype(jnp.bfloat16)
    h = jnp.dot(xb, w1, preferred_element_type=jnp.float32)
    b1v = b1_ref[pl.ds(e, 1), pl.ds(fb * FB, FB)]
    h = jnp.maximum(h + b1v, 0.0)
    w2 = w2_ref[0].astype(jnp.bfloat16)
    contrib = jnp.dot(
        h.astype(jnp.bfloat16), w2, preferred_element_type=jnp.float32
    )

    @pl.when(fb == 0)
    def _():
        out_ref[...] = contrib + b2_ref[pl.ds(e, 1), :]

    @pl.when(fb > 0)
    def _():
        out_ref[...] = out_ref[...] + contrib


def _ffn(buf, W1, b1, W2, b2):
    n_cb = C // CB
    n_fb = F // FB
    return pl.pallas_call(
        _ffn_body,
        grid=(E, n_cb, n_fb),
        in_specs=[
            pl.BlockSpec((CB, D), lambda e, cb, fb: (e * (C // CB) + cb, 0)),
            pl.BlockSpec((1, D, FB), lambda e, cb, fb: (e, 0, fb)),
            pl.BlockSpec((E, F), lambda e, cb, fb: (0, 0)),
            pl.BlockSpec((1, FB, D), lambda e, cb, fb: (e, fb, 0)),
            pl.BlockSpec((E, D), lambda e, cb, fb: (0, 0)),
        ],
        out_specs=pl.BlockSpec((CB, D), lambda e, cb, fb: (e * (C // CB) + cb, 0)),
        out_shape=jax.ShapeDtypeStruct((S_TOT, D), jnp.float32),
    )(buf, W1, b1, W2, b2)


def _combine_body(r0_ref, r1_ref, g0_ref, g1_ref, y_ref):
    g0 = g0_ref[:, 0:1]
    g1 = g1_ref[:, 0:1]
    y_ref[...] = g0 * r0_ref[...] + g1 * r1_ref[...]


def _combine(r0, r1, g0, g1):
    return pl.pallas_call(
        _combine_body,
        grid=(NT,),
        in_specs=[
            pl.BlockSpec((TB, D), lambda i: (i, 0)),
            pl.BlockSpec((TB, D), lambda i: (i, 0)),
            pl.BlockSpec((TB, 8), lambda i: (i, 0)),
            pl.BlockSpec((TB, 8), lambda i: (i, 0)),
        ],
        out_specs=pl.BlockSpec((TB, D), lambda i: (i, 0)),
        out_shape=jax.ShapeDtypeStruct((T, D), jnp.float32),
    )(r0, r1, g0, g1)


def kernel(input, Wr, W1, b1, W2, b2):
    x = input
    wr_p = jnp.pad(Wr, ((0, 0), (0, LANES - E)))
    build_inv, dispatch, combine_gather = _sc_kernels()
    d0, d1, c0, c1, g0, g1, loss = _router(x, wr_p)
    inv = build_inv(d0[:, 0], d1[:, 0])
    buf = dispatch(x, inv)
    out = _ffn(buf, W1, b1, W2, b2)
    r0, r1 = combine_gather(out, c0[:, 0], c1[:, 0])
    y = _combine(r0, r1, g0, g1)
    return y, loss[0, 0]


# speedup vs baseline: 1.2265x; 1.2265x over previous
import functools

import jax
import jax.numpy as jnp
from jax import lax
from jax.experimental import pallas as pl
from jax.experimental.pallas import tpu as pltpu
from jax.experimental.pallas import tpu_sc as plsc

T = 4096
D = 768
E = 8
K = 2
F = 3072
C = 1280
S_TOT = E * C
TB = 512
NT = T // TB
CB = 256
FB = 768
NW = 32
LANES = 128


def _router_body(x_ref, wr_ref, d0_ref, d1_ref, c0_ref, c1_ref, g0_ref,
                 g1_ref, loss_ref, me_acc, cnt_acc):
    i = pl.program_id(0)

    @pl.when(i == 0)
    def _():
        me_acc[...] = jnp.zeros((8, LANES), jnp.float32)
        cnt_acc[...] = jnp.zeros((8, LANES), jnp.float32)

    x = x_ref[...]
    logits = jnp.dot(x, wr_ref[...], preferred_element_type=jnp.float32)
    lane = lax.broadcasted_iota(jnp.int32, (TB, LANES), 1)
    emask = lane < E
    lm = jnp.where(emask, logits, jnp.float32(-1e30))
    m = jnp.max(lm, axis=1, keepdims=True)
    ex = jnp.where(emask, jnp.exp(lm - m), 0.0)
    probs = ex / jnp.sum(ex, axis=1, keepdims=True)

    v1 = jnp.max(probs, axis=1, keepdims=True)
    i1 = jnp.min(jnp.where(probs == v1, lane, LANES), axis=1, keepdims=True)
    oh1 = (lane == i1).astype(jnp.float32)
    probs2 = jnp.where(lane == i1, -1.0, probs)
    v2 = jnp.max(probs2, axis=1, keepdims=True)
    i2 = jnp.min(jnp.where(probs2 == v2, lane, LANES), axis=1, keepdims=True)
    oh2 = (lane == i2).astype(jnp.float32)
    den = v1 + v2
    g1v = v1 / den
    g2v = v2 / den

    oh = oh1 + oh2
    r_io = lax.broadcasted_iota(jnp.int32, (TB, TB), 0)
    c_io = lax.broadcasted_iota(jnp.int32, (TB, TB), 1)
    tri = (r_io > c_io).astype(jnp.float32)
    pref = jnp.dot(tri, oh, preferred_element_type=jnp.float32)
    carry = cnt_acc[0:1, :]
    p_all = carry + pref
    pos0 = jnp.sum(jnp.where(lane == i1, p_all, 0.0), axis=1, keepdims=True)
    pos1 = jnp.sum(jnp.where(lane == i2, p_all, 0.0), axis=1, keepdims=True)
    pos0 = pos0.astype(jnp.int32)
    pos1 = pos1.astype(jnp.int32)
    keep0 = pos0 < C
    keep1 = pos1 < C
    slot0 = i1 * C + pos0
    slot1 = i2 * C + pos1

    d0 = jnp.where(keep0, slot0, S_TOT)
    d1 = jnp.where(keep1, slot1, S_TOT)
    c0 = jnp.where(keep0, slot0, 0)
    c1 = jnp.where(keep1, slot1, 0)
    cg0 = jnp.where(keep0, g1v, 0.0)
    cg1 = jnp.where(keep1, g2v, 0.0)

    d0_ref[...] = jnp.broadcast_to(d0, (TB, 8))
    d1_ref[...] = jnp.broadcast_to(d1, (TB, 8))
    c0_ref[...] = jnp.broadcast_to(c0, (TB, 8))
    c1_ref[...] = jnp.broadcast_to(c1, (TB, 8))
    g0_ref[...] = jnp.broadcast_to(cg0, (TB, 8))
    g1_ref[...] = jnp.broadcast_to(cg1, (TB, 8))

    cnt_acc[0:1, :] = carry + jnp.sum(oh, axis=0, keepdims=True)
    me_acc[0:1, :] = me_acc[0:1, :] + jnp.sum(probs, axis=0, keepdims=True)

    @pl.when(i == NT - 1)
    def _():
        me = me_acc[0:1, :] / T
        ce = cnt_acc[0:1, :] / (T * K)
        loss = E * jnp.sum(me * ce)
        loss_ref[...] = jnp.zeros((8, LANES), jnp.float32) + loss


def _router(x, wr_p):
    return pl.pallas_call(
        _router_body,
        grid=(NT,),
        in_specs=[
            pl.BlockSpec((TB, D), lambda i: (i, 0)),
            pl.BlockSpec((D, LANES), lambda i: (0, 0)),
        ],
        out_specs=[
            pl.BlockSpec((TB, 8), lambda i: (i, 0)),
            pl.BlockSpec((TB, 8), lambda i: (i, 0)),
            pl.BlockSpec((TB, 8), lambda i: (i, 0)),
            pl.BlockSpec((TB, 8), lambda i: (i, 0)),
            pl.BlockSpec((TB, 8), lambda i: (i, 0)),
            pl.BlockSpec((TB, 8), lambda i: (i, 0)),
            pl.BlockSpec((8, LANES), lambda i: (0, 0)),
        ],
        out_shape=[
            jax.ShapeDtypeStruct((T, 8), jnp.int32),
            jax.ShapeDtypeStruct((T, 8), jnp.int32),
            jax.ShapeDtypeStruct((T, 8), jnp.int32),
            jax.ShapeDtypeStruct((T, 8), jnp.int32),
            jax.ShapeDtypeStruct((T, 8), jnp.float32),
            jax.ShapeDtypeStruct((T, 8), jnp.float32),
            jax.ShapeDtypeStruct((8, LANES), jnp.float32),
        ],
        scratch_shapes=[
            pltpu.VMEM((8, LANES), jnp.float32),
            pltpu.VMEM((8, LANES), jnp.float32),
        ],
    )(x, wr_p)


_INV_N = S_TOT + 128
_DG_PW = S_TOT // NW
_DG_CH = 64
_CG_PW = T // NW
_CG_CH = 64


@functools.cache
def _sc_kernels():
    mesh = plsc.VectorSubcoreMesh(core_axis_name="c", subcore_axis_name="s")

    @functools.partial(
        pl.kernel,
        mesh=mesh,
        out_type=jax.ShapeDtypeStruct((_INV_N,), jnp.int32),
        scratch_types=[
            pltpu.VMEM((T,), jnp.int32),
            pltpu.VMEM((T,), jnp.int32),
            pltpu.VMEM((_INV_N,), jnp.int32),
        ],
        compiler_params=pltpu.CompilerParams(needs_layout_passes=False),
    )
    def build_inv(d0_hbm, d1_hbm, inv_hbm, s0_v, s1_v, inv_v):
        wid = lax.axis_index("s") * 2 + lax.axis_index("c")

        @pl.when(wid == 0)
        def _():
            pltpu.sync_copy(d0_hbm, s0_v)
            pltpu.sync_copy(d1_hbm, s1_v)

            def init_body(j, carry):
                inv_v[pl.ds(j * 16, 16)] = jnp.zeros((16,), jnp.int32)
                return carry

            lax.fori_loop(0, _INV_N // 16, init_body, 0)

            def scat_body(j, carry):
                base = j * 16
                toks = lax.iota(jnp.int32, 16) + base
                plsc.store_scatter(inv_v, [s0_v[pl.ds(base, 16)]], toks)
                plsc.store_scatter(inv_v, [s1_v[pl.ds(base, 16)]], toks)
                return carry

            lax.fori_loop(0, T // 16, scat_body, 0)
            pltpu.sync_copy(inv_v, inv_hbm)

    @functools.partial(
        pl.kernel,
        mesh=mesh,
        out_type=jax.ShapeDtypeStruct((S_TOT, D), jnp.float32),
        scratch_types=[
            pltpu.VMEM((_DG_PW,), jnp.int32),
            pltpu.VMEM((2, _DG_CH, D), jnp.float32),
            pltpu.SemaphoreType.DMA,
            pltpu.SemaphoreType.DMA,
        ],
    )
    def dispatch(x_hbm, inv_hbm, buf_hbm, idx_v, rows_v, sem0, sem1):
        wid = lax.axis_index("s") * 2 + lax.axis_index("c")
        base = wid * _DG_PW
        pltpu.sync_copy(inv_hbm.at[pl.ds(base, _DG_PW)], idx_v)
        nch = _DG_PW // _DG_CH
        sems = (sem0, sem1)
        cps = [None] * nch

        def start(ch):
            return pltpu.async_copy(
                x_hbm.at[idx_v.at[pl.ds(ch * _DG_CH, _DG_CH)]],
                rows_v.at[ch % 2],
                sems[ch % 2],
            )

        cps[0] = start(0)
        for ch in range(nch):
            if ch + 1 < nch:
                cps[ch + 1] = start(ch + 1)
            cps[ch].wait()
            pltpu.sync_copy(
                rows_v.at[ch % 2],
                buf_hbm.at[pl.ds(base + ch * _DG_CH, _DG_CH)],
            )

    @functools.partial(
        pl.kernel,
        mesh=mesh,
        out_type=(
            jax.ShapeDtypeStruct((T, D), jnp.float32),
            jax.ShapeDtypeStruct((T, D), jnp.float32),
        ),
        scratch_types=[
            pltpu.VMEM((_CG_PW,), jnp.int32),
            pltpu.VMEM((_CG_PW,), jnp.int32),
            pltpu.VMEM((2, _CG_CH, D), jnp.float32),
            pltpu.SemaphoreType.DMA,
            pltpu.SemaphoreType.DMA,
        ],
    )
    def combine_gather(out_hbm, c0_hbm, c1_hbm, r0_hbm, r1_hbm, i0_v, i1_v,
                       rows_v, sem0, sem1):
        wid = lax.axis_index("s") * 2 + lax.axis_index("c")
        base = wid * _CG_PW
        pltpu.sync_copy(c0_hbm.at[pl.ds(base, _CG_PW)], i0_v)
        pltpu.sync_copy(c1_hbm.at[pl.ds(base, _CG_PW)], i1_v)
        sems = (sem0, sem1)
        tasks = []
        for iv, out in ((i0_v, r0_hbm), (i1_v, r1_hbm)):
            for ch in range(_CG_PW // _CG_CH):
                tasks.append((iv, ch * _CG_CH, out))
        cps = [None] * len(tasks)

        def start(k):
            iv, off, _ = tasks[k]
            return pltpu.async_copy(
                out_hbm.at[iv.at[pl.ds(off, _CG_CH)]],
                rows_v.at[k % 2],
                sems[k % 2],
            )

        cps[0] = start(0)
        for k in range(len(tasks)):
            if k + 1 < len(tasks):
                cps[k + 1] = start(k + 1)
            cps[k].wait()
            _, off, out = tasks[k]
            pltpu.sync_copy(
                rows_v.at[k % 2], out.at[pl.ds(base + off, _CG_CH)]
            )

    return build_inv, dispatch, combine_gather


def _ffn_body(buf_ref, w1_ref, b1_ref, w2_ref, b2_ref, out_ref):
    e = pl.program_id(0)
    fb = pl.program_id(1)
    xb = buf_ref[...].astype(jnp.bfloat16)
    h = jnp.dot(xb, w1_ref[0], preferred_element_type=jnp.float32)
    b1v = b1_ref[pl.ds(e, 1), pl.ds(fb * FB, FB)]
    h = jnp.maximum(h + b1v, 0.0)
    contrib = jnp.dot(
        h.astype(jnp.bfloat16), w2_ref[0], preferred_element_type=jnp.float32
    )

    @pl.when(fb == 0)
    def _():
        out_ref[...] = contrib + b2_ref[pl.ds(e, 1), :]

    @pl.when(fb > 0)
    def _():
        out_ref[...] = out_ref[...] + contrib


def _ffn(buf, W1, b1, W2, b2):
    n_fb = F // FB
    return pl.pallas_call(
        _ffn_body,
        grid=(E, n_fb),
        in_specs=[
            pl.BlockSpec((C, D), lambda e, fb: (e, 0)),
            pl.BlockSpec((1, D, FB), lambda e, fb: (e, 0, fb)),
            pl.BlockSpec((E, F), lambda e, fb: (0, 0)),
            pl.BlockSpec((1, FB, D), lambda e, fb: (e, fb, 0)),
            pl.BlockSpec((E, D), lambda e, fb: (0, 0)),
        ],
        out_specs=pl.BlockSpec((C, D), lambda e, fb: (e, 0)),
        out_shape=jax.ShapeDtypeStruct((S_TOT, D), jnp.float32),
    )(buf, W1.astype(jnp.bfloat16), b1, W2.astype(jnp.bfloat16), b2)


def _combine_body(r0_ref, r1_ref, g0_ref, g1_ref, y_ref):
    g0 = g0_ref[:, 0:1]
    g1 = g1_ref[:, 0:1]
    y_ref[...] = g0 * r0_ref[...] + g1 * r1_ref[...]


def _combine(r0, r1, g0, g1):
    return pl.pallas_call(
        _combine_body,
        grid=(NT,),
        in_specs=[
            pl.BlockSpec((TB, D), lambda i: (i, 0)),
            pl.BlockSpec((TB, D), lambda i: (i, 0)),
            pl.BlockSpec((TB, 8), lambda i: (i, 0)),
            pl.BlockSpec((TB, 8), lambda i: (i, 0)),
        ],
        out_specs=pl.BlockSpec((TB, D), lambda i: (i, 0)),
        out_shape=jax.ShapeDtypeStruct((T, D), jnp.float32),
    )(r0, r1, g0, g1)


def kernel(input, Wr, W1, b1, W2, b2):
    x = input
    wr_p = jnp.pad(Wr, ((0, 0), (0, LANES - E)))
    build_inv, dispatch, combine_gather = _sc_kernels()
    d0, d1, c0, c1, g0, g1, loss = _router(x, wr_p)
    inv = build_inv(d0[:, 0], d1[:, 0])
    buf = dispatch(x, inv)
    out = _ffn(buf, W1, b1, W2, b2)
    r0, r1 = combine_gather(out, c0[:, 0], c1[:, 0])
    y = _combine(r0, r1, g0, g1)
    return y, loss[0, 0]


# speedup vs baseline: 1.6975x; 1.3840x over previous
import functools

import jax
import jax.numpy as jnp
from jax import lax
from jax.experimental import pallas as pl
from jax.experimental.pallas import tpu as pltpu
from jax.experimental.pallas import tpu_sc as plsc

T = 4096
D = 768
E = 8
K = 2
F = 3072
C = 1280
S_TOT = E * C
TB = 512
NT = T // TB
CB = 256
FB = 768
NW = 32
LANES = 128


def _router_body(x_ref, wr_ref, d0_ref, d1_ref, c0_ref, c1_ref, g0_ref,
                 g1_ref, loss_ref, me_acc, cnt_acc):
    i = pl.program_id(0)

    @pl.when(i == 0)
    def _():
        me_acc[...] = jnp.zeros((8, LANES), jnp.float32)
        cnt_acc[...] = jnp.zeros((8, LANES), jnp.float32)

    x = x_ref[...]
    logits = jnp.dot(x, wr_ref[...], preferred_element_type=jnp.float32)
    lane = lax.broadcasted_iota(jnp.int32, (TB, LANES), 1)
    emask = lane < E
    lm = jnp.where(emask, logits, jnp.float32(-1e30))
    m = jnp.max(lm, axis=1, keepdims=True)
    ex = jnp.where(emask, jnp.exp(lm - m), 0.0)
    probs = ex / jnp.sum(ex, axis=1, keepdims=True)

    v1 = jnp.max(probs, axis=1, keepdims=True)
    i1 = jnp.min(jnp.where(probs == v1, lane, LANES), axis=1, keepdims=True)
    oh1 = (lane == i1).astype(jnp.float32)
    probs2 = jnp.where(lane == i1, -1.0, probs)
    v2 = jnp.max(probs2, axis=1, keepdims=True)
    i2 = jnp.min(jnp.where(probs2 == v2, lane, LANES), axis=1, keepdims=True)
    oh2 = (lane == i2).astype(jnp.float32)
    den = v1 + v2
    g1v = v1 / den
    g2v = v2 / den

    oh = oh1 + oh2
    r_io = lax.broadcasted_iota(jnp.int32, (TB, TB), 0)
    c_io = lax.broadcasted_iota(jnp.int32, (TB, TB), 1)
    tri = (r_io > c_io).astype(jnp.float32)
    pref = jnp.dot(tri, oh, preferred_element_type=jnp.float32)
    carry = cnt_acc[0:1, :]
    p_all = carry + pref
    pos0 = jnp.sum(jnp.where(lane == i1, p_all, 0.0), axis=1, keepdims=True)
    pos1 = jnp.sum(jnp.where(lane == i2, p_all, 0.0), axis=1, keepdims=True)
    pos0 = pos0.astype(jnp.int32)
    pos1 = pos1.astype(jnp.int32)
    keep0 = pos0 < C
    keep1 = pos1 < C
    slot0 = i1 * C + pos0
    slot1 = i2 * C + pos1

    d0 = jnp.where(keep0, slot0, S_TOT)
    d1 = jnp.where(keep1, slot1, S_TOT)
    c0 = jnp.where(keep0, slot0, 0)
    c1 = jnp.where(keep1, slot1, 0)
    cg0 = jnp.where(keep0, g1v, 0.0)
    cg1 = jnp.where(keep1, g2v, 0.0)

    d0_ref[...] = jnp.broadcast_to(d0, (TB, 8))
    d1_ref[...] = jnp.broadcast_to(d1, (TB, 8))
    c0_ref[...] = jnp.broadcast_to(c0, (TB, 8))
    c1_ref[...] = jnp.broadcast_to(c1, (TB, 8))
    g0_ref[...] = jnp.broadcast_to(cg0, (TB, 8))
    g1_ref[...] = jnp.broadcast_to(cg1, (TB, 8))

    cnt_acc[0:1, :] = carry + jnp.sum(oh, axis=0, keepdims=True)
    me_acc[0:1, :] = me_acc[0:1, :] + jnp.sum(probs, axis=0, keepdims=True)

    @pl.when(i == NT - 1)
    def _():
        me = me_acc[0:1, :] / T
        ce = cnt_acc[0:1, :] / (T * K)
        loss = E * jnp.sum(me * ce)
        loss_ref[...] = jnp.zeros((8, LANES), jnp.float32) + loss


def _router(x, wr_p):
    return pl.pallas_call(
        _router_body,
        grid=(NT,),
        in_specs=[
            pl.BlockSpec((TB, D), lambda i: (i, 0)),
            pl.BlockSpec((D, LANES), lambda i: (0, 0)),
        ],
        out_specs=[
            pl.BlockSpec((TB, 8), lambda i: (i, 0)),
            pl.BlockSpec((TB, 8), lambda i: (i, 0)),
            pl.BlockSpec((TB, 8), lambda i: (i, 0)),
            pl.BlockSpec((TB, 8), lambda i: (i, 0)),
            pl.BlockSpec((TB, 8), lambda i: (i, 0)),
            pl.BlockSpec((TB, 8), lambda i: (i, 0)),
            pl.BlockSpec((8, LANES), lambda i: (0, 0)),
        ],
        out_shape=[
            jax.ShapeDtypeStruct((T, 8), jnp.int32),
            jax.ShapeDtypeStruct((T, 8), jnp.int32),
            jax.ShapeDtypeStruct((T, 8), jnp.int32),
            jax.ShapeDtypeStruct((T, 8), jnp.int32),
            jax.ShapeDtypeStruct((T, 8), jnp.float32),
            jax.ShapeDtypeStruct((T, 8), jnp.float32),
            jax.ShapeDtypeStruct((8, LANES), jnp.float32),
        ],
        scratch_shapes=[
            pltpu.VMEM((8, LANES), jnp.float32),
            pltpu.VMEM((8, LANES), jnp.float32),
        ],
    )(x, wr_p)


_INV_N = S_TOT + 128
_DG_PW = S_TOT // NW
_DG_CH = 64
_CG_PW = T // NW
_CG_CH = 64


@functools.cache
def _sc_kernels():
    mesh = plsc.VectorSubcoreMesh(core_axis_name="c", subcore_axis_name="s")

    @functools.partial(
        pl.kernel,
        mesh=mesh,
        out_type=jax.ShapeDtypeStruct((_INV_N,), jnp.int32),
        scratch_types=[
            pltpu.VMEM((T,), jnp.int32),
            pltpu.VMEM((T,), jnp.int32),
            pltpu.VMEM((_INV_N,), jnp.int32),
        ],
        compiler_params=pltpu.CompilerParams(needs_layout_passes=False),
    )
    def build_inv(d0_hbm, d1_hbm, inv_hbm, s0_v, s1_v, inv_v):
        wid = lax.axis_index("s") * 2 + lax.axis_index("c")

        @pl.when(wid == 0)
        def _():
            pltpu.sync_copy(d0_hbm, s0_v)
            pltpu.sync_copy(d1_hbm, s1_v)

            def init_body(j, carry):
                fill = (lax.iota(jnp.int32, 16) + j * 16) & (T - 1)
                inv_v[pl.ds(j * 16, 16)] = fill
                return carry

            lax.fori_loop(0, _INV_N // 16, init_body, 0)

            def scat_body(j, carry):
                base = j * 16
                toks = lax.iota(jnp.int32, 16) + base
                plsc.store_scatter(inv_v, [s0_v[pl.ds(base, 16)]], toks)
                plsc.store_scatter(inv_v, [s1_v[pl.ds(base, 16)]], toks)
                return carry

            lax.fori_loop(0, T // 16, scat_body, 0)
            pltpu.sync_copy(inv_v, inv_hbm)

    @functools.partial(
        pl.kernel,
        mesh=mesh,
        out_type=jax.ShapeDtypeStruct((S_TOT, D), jnp.float32),
        scratch_types=[
            pltpu.VMEM((_DG_PW,), jnp.int32),
            pltpu.VMEM((2, _DG_CH, D), jnp.float32),
            pltpu.SemaphoreType.DMA,
            pltpu.SemaphoreType.DMA,
        ],
    )
    def dispatch(x_hbm, inv_hbm, buf_hbm, idx_v, rows_v, sem0, sem1):
        wid = lax.axis_index("s") * 2 + lax.axis_index("c")
        base = wid * _DG_PW
        pltpu.sync_copy(inv_hbm.at[pl.ds(base, _DG_PW)], idx_v)
        nch = _DG_PW // _DG_CH
        sems = (sem0, sem1)
        cps = [None] * nch

        def start(ch):
            return pltpu.async_copy(
                x_hbm.at[idx_v.at[pl.ds(ch * _DG_CH, _DG_CH)]],
                rows_v.at[ch % 2],
                sems[ch % 2],
            )

        cps[0] = start(0)
        for ch in range(nch):
            if ch + 1 < nch:
                cps[ch + 1] = start(ch + 1)
            cps[ch].wait()
            pltpu.sync_copy(
                rows_v.at[ch % 2],
                buf_hbm.at[pl.ds(base + ch * _DG_CH, _DG_CH)],
            )

    @functools.partial(
        pl.kernel,
        mesh=mesh,
        out_type=(
            jax.ShapeDtypeStruct((T, D), jnp.float32),
            jax.ShapeDtypeStruct((T, D), jnp.float32),
        ),
        scratch_types=[
            pltpu.VMEM((_CG_PW,), jnp.int32),
            pltpu.VMEM((_CG_PW,), jnp.int32),
            pltpu.VMEM((2, _CG_CH, D), jnp.float32),
            pltpu.SemaphoreType.DMA,
            pltpu.SemaphoreType.DMA,
        ],
    )
    def combine_gather(out_hbm, c0_hbm, c1_hbm, r0_hbm, r1_hbm, i0_v, i1_v,
                       rows_v, sem0, sem1):
        wid = lax.axis_index("s") * 2 + lax.axis_index("c")
        base = wid * _CG_PW
        pltpu.sync_copy(c0_hbm.at[pl.ds(base, _CG_PW)], i0_v)
        pltpu.sync_copy(c1_hbm.at[pl.ds(base, _CG_PW)], i1_v)
        sems = (sem0, sem1)
        tasks = []
        for iv, out in ((i0_v, r0_hbm), (i1_v, r1_hbm)):
            for ch in range(_CG_PW // _CG_CH):
                tasks.append((iv, ch * _CG_CH, out))
        cps = [None] * len(tasks)

        def start(k):
            iv, off, _ = tasks[k]
            return pltpu.async_copy(
                out_hbm.at[iv.at[pl.ds(off, _CG_CH)]],
                rows_v.at[k % 2],
                sems[k % 2],
            )

        cps[0] = start(0)
        for k in range(len(tasks)):
            if k + 1 < len(tasks):
                cps[k + 1] = start(k + 1)
            cps[k].wait()
            _, off, out = tasks[k]
            pltpu.sync_copy(
                rows_v.at[k % 2], out.at[pl.ds(base + off, _CG_CH)]
            )

    return build_inv, dispatch, combine_gather


def _ffn_body(buf_ref, w1_ref, b1_ref, w2_ref, b2_ref, out_ref):
    e = pl.program_id(0)
    fb = pl.program_id(1)
    xb = buf_ref[...].astype(jnp.bfloat16)
    h = jnp.dot(xb, w1_ref[0], preferred_element_type=jnp.float32)
    b1v = b1_ref[pl.ds(e, 1), pl.ds(fb * FB, FB)]
    h = jnp.maximum(h + b1v, 0.0)
    contrib = jnp.dot(
        h.astype(jnp.bfloat16), w2_ref[0], preferred_element_type=jnp.float32
    )

    @pl.when(fb == 0)
    def _():
        out_ref[...] = contrib + b2_ref[pl.ds(e, 1), :]

    @pl.when(fb > 0)
    def _():
        out_ref[...] = out_ref[...] + contrib


def _ffn(buf, W1, b1, W2, b2):
    n_fb = F // FB
    return pl.pallas_call(
        _ffn_body,
        grid=(E, n_fb),
        in_specs=[
            pl.BlockSpec((C, D), lambda e, fb: (e, 0)),
            pl.BlockSpec((1, D, FB), lambda e, fb: (e, 0, fb)),
            pl.BlockSpec((E, F), lambda e, fb: (0, 0)),
            pl.BlockSpec((1, FB, D), lambda e, fb: (e, fb, 0)),
            pl.BlockSpec((E, D), lambda e, fb: (0, 0)),
        ],
        out_specs=pl.BlockSpec((C, D), lambda e, fb: (e, 0)),
        out_shape=jax.ShapeDtypeStruct((S_TOT, D), jnp.float32),
    )(buf, W1.astype(jnp.bfloat16), b1, W2.astype(jnp.bfloat16), b2)


def _combine_body(r0_ref, r1_ref, g0_ref, g1_ref, y_ref):
    g0 = g0_ref[:, 0:1]
    g1 = g1_ref[:, 0:1]
    y_ref[...] = g0 * r0_ref[...] + g1 * r1_ref[...]


def _combine(r0, r1, g0, g1):
    return pl.pallas_call(
        _combine_body,
        grid=(NT,),
        in_specs=[
            pl.BlockSpec((TB, D), lambda i: (i, 0)),
            pl.BlockSpec((TB, D), lambda i: (i, 0)),
            pl.BlockSpec((TB, 8), lambda i: (i, 0)),
            pl.BlockSpec((TB, 8), lambda i: (i, 0)),
        ],
        out_specs=pl.BlockSpec((TB, D), lambda i: (i, 0)),
        out_shape=jax.ShapeDtypeStruct((T, D), jnp.float32),
    )(r0, r1, g0, g1)


def kernel(input, Wr, W1, b1, W2, b2):
    x = input
    wr_p = jnp.pad(Wr, ((0, 0), (0, LANES - E)))
    build_inv, dispatch, combine_gather = _sc_kernels()
    d0, d1, c0, c1, g0, g1, loss = _router(x, wr_p)
    inv = build_inv(d0[:, 0], d1[:, 0])
    buf = dispatch(x, inv)
    out = _ffn(buf, W1, b1, W2, b2)
    r0, r1 = combine_gather(out, c0[:, 0], c1[:, 0])
    y = _combine(r0, r1, g0, g1)
    return y, loss[0, 0]
